# baseline (speedup 1.0000x reference)
import jax
import jax.numpy as jnp
from jax.experimental import pallas as pl

_B, _C, _H, _W = 8, 256, 64, 64
_HW = _H * _W
_P = 100
_PPAD = 128
_LPAD = 128


def _proposal_kernel(xp_ref, w1_ref, b1_ref, w2_ref, b2_ref,
                     scores_ref, bbox_ref, feats_ref, loc_ref):
    xp = xp_ref[0]
    lane = jax.lax.broadcasted_iota(jnp.int32, (1, _HW), 1)
    xcol = lane % _W

    acc = jnp.zeros((_C, _HW), jnp.float32)
    for k in range(9):
        dy, dx = k // 3 - 1, k % 3 - 1
        start = _LPAD + dy * _W + dx
        xs = xp[:, start:start + _HW]
        if dx == -1:
            xs = xs * (xcol >= 1).astype(jnp.float32)
        elif dx == 1:
            xs = xs * (xcol <= _W - 2).astype(jnp.float32)
        acc = acc + jax.lax.dot_general(
            w1_ref[k], xs, (((1,), (0,)), ((), ())),
            preferred_element_type=jnp.float32)
    hid = jnp.maximum(acc + b1_ref[:, 0:1], 0.0)

    head2 = jax.lax.dot_general(
        w2_ref[...], hid, (((1,), (0,)), ((), ())),
        preferred_element_type=jnp.float32) + b2_ref[:, 0:1]
    scores = jax.nn.sigmoid(head2[0:1, :])

    lane_p = jax.lax.broadcasted_iota(jnp.int32, (1, _PPAD), 1)
    sub_p = jax.lax.broadcasted_iota(jnp.int32, (_PPAD, 1), 0)

    def body(t, carry):
        sc, vals, idxr, idxc = carry
        m = jnp.max(sc)
        i = jnp.min(jnp.where(sc == m, lane, jnp.int32(_HW)))
        vals = jnp.where(lane_p == t, m, vals)
        idxr = jnp.where(lane_p == t, i, idxr)
        idxc = jnp.where(sub_p == t, i, idxc)
        sc = jnp.where(lane == i, -1.0, sc)
        return sc, vals, idxr, idxc

    carry0 = (scores,
              jnp.zeros((1, _PPAD), jnp.float32),
              jnp.full((1, _PPAD), -1, jnp.int32),
              jnp.full((_PPAD, 1), -1, jnp.int32))
    _, vals, idxr, idxc = jax.lax.fori_loop(0, _P, body, carry0)

    onehot = (idxc == lane).astype(jnp.float32)
    xc = xp[:, _LPAD:_LPAD + _HW]
    fsel = jax.lax.dot_general(
        onehot, xc, (((1,), (1,)), ((), ())),
        preferred_element_type=jnp.float32)
    dsel = jax.lax.dot_general(
        head2, onehot, (((1,), (1,)), ((), ())),
        preferred_element_type=jnp.float32)

    xx = idxr % _W
    yy = idxr // _W
    xxf = xx.astype(jnp.float32)
    yyf = yy.astype(jnp.float32)
    dxv, dyv, dwv, dhv = dsel[1:2], dsel[2:3], dsel[3:4], dsel[4:5]
    cx = xxf * 8.0 + dxv * 8.0
    cy = yyf * 8.0 + dyv * 8.0
    wv = jnp.exp(dwv) * 8.0
    hv = jnp.exp(dhv) * 8.0

    scores_ref[0, 0:1, :] = vals
    bbox_ref[0, 0:1, :] = cx - wv * 0.5
    bbox_ref[0, 1:2, :] = cy - hv * 0.5
    bbox_ref[0, 2:3, :] = cx + wv * 0.5
    bbox_ref[0, 3:4, :] = cy + hv * 0.5
    bbox_ref[0, 4:8, :] = jnp.zeros((4, _PPAD), jnp.float32)
    scores_ref[0, 1:8, :] = jnp.zeros((7, _PPAD), jnp.float32)
    feats_ref[0] = fsel
    loc_ref[0, 0:1, :] = xx
    loc_ref[0, 1:2, :] = yy
    loc_ref[0, 2:8, :] = jnp.zeros((6, _PPAD), jnp.int32)


def kernel(features, obj_w1, obj_b1, obj_w2, obj_b2,
           box_w1, box_b1, box_w2, box_b2):
    xf = features.reshape(_B, _C, _HW)
    xp = jnp.pad(xf, ((0, 0), (0, 0), (_LPAD, _LPAD)))

    w1c = jnp.concatenate([obj_w1, box_w1], axis=0)
    w1t = jnp.transpose(w1c, (2, 3, 0, 1)).reshape(9, _C, _C)
    b1c = jnp.concatenate([obj_b1, box_b1], axis=0).reshape(_C, 1)
    b1p = jnp.broadcast_to(b1c, (_C, 128))

    w2 = jnp.zeros((8, _C), jnp.float32)
    w2 = w2.at[0, :128].set(obj_w2.reshape(128))
    w2 = w2.at[1:5, 128:].set(box_w2.reshape(4, 128))
    b2 = jnp.zeros((8,), jnp.float32)
    b2 = b2.at[0].set(obj_b2[0]).at[1:5].set(box_b2)
    b2p = jnp.broadcast_to(b2.reshape(8, 1), (8, 128))

    scores_o, bbox_o, feats_o, loc_o = pl.pallas_call(
        _proposal_kernel,
        grid=(_B,),
        in_specs=[
            pl.BlockSpec((1, _C, _HW + 2 * _LPAD), lambda b: (b, 0, 0)),
            pl.BlockSpec((9, _C, _C), lambda b: (0, 0, 0)),
            pl.BlockSpec((_C, 128), lambda b: (0, 0)),
            pl.BlockSpec((8, _C), lambda b: (0, 0)),
            pl.BlockSpec((8, 128), lambda b: (0, 0)),
        ],
        out_specs=[
            pl.BlockSpec((1, 8, _PPAD), lambda b: (b, 0, 0)),
            pl.BlockSpec((1, 8, _PPAD), lambda b: (b, 0, 0)),
            pl.BlockSpec((1, _PPAD, _C), lambda b: (b, 0, 0)),
            pl.BlockSpec((1, 8, _PPAD), lambda b: (b, 0, 0)),
        ],
        out_shape=[
            jax.ShapeDtypeStruct((_B, 8, _PPAD), jnp.float32),
            jax.ShapeDtypeStruct((_B, 8, _PPAD), jnp.float32),
            jax.ShapeDtypeStruct((_B, _PPAD, _C), jnp.float32),
            jax.ShapeDtypeStruct((_B, 8, _PPAD), jnp.int32),
        ],
    )(xp, w1t, b1p, w2, b2p)

    top_scores = scores_o[:, 0, :_P]
    bboxes = bbox_o[:, :4, :_P].transpose(0, 2, 1)
    feats = feats_o[:, :_P, :]
    locations = loc_o[:, :2, :_P].transpose(0, 2, 1)
    return bboxes, top_scores, feats, locations


# speedup vs baseline: 2.4073x; 2.4073x over previous
import jax
import jax.numpy as jnp
from jax.experimental import pallas as pl

_B, _C, _H, _W = 8, 256, 64, 64
_HW = _H * _W
_P = 100
_PPAD = 128
_LPAD = 128


def _heads_kernel(xp_ref, w1_ref, b1_ref, w2_ref, b2_ref,
                  scores_ref, head2_ref):
    xp = xp_ref[0]
    lane = jax.lax.broadcasted_iota(jnp.int32, (1, _HW), 1)
    xcol = lane % _W

    acc = jnp.zeros((_C, _HW), jnp.float32)
    for k in range(9):
        dy, dx = k // 3 - 1, k % 3 - 1
        start = _LPAD + dy * _W + dx
        xs = xp[:, start:start + _HW]
        if dx == -1:
            xs = xs * (xcol >= 1).astype(jnp.float32)
        elif dx == 1:
            xs = xs * (xcol <= _W - 2).astype(jnp.float32)
        acc = acc + jax.lax.dot_general(
            w1_ref[k], xs, (((1,), (0,)), ((), ())),
            preferred_element_type=jnp.float32)
    hid = jnp.maximum(acc + b1_ref[:, 0:1], 0.0)

    head2 = jax.lax.dot_general(
        w2_ref[...], hid, (((1,), (0,)), ((), ())),
        preferred_element_type=jnp.float32) + b2_ref[:, 0:1]
    scores_ref[0] = jax.nn.sigmoid(head2[0:1, :])
    head2_ref[0] = head2


def _topk_kernel(scores_ref, vals_ref, idx_ref):
    sc = scores_ref[:, 0, :]
    lane = jax.lax.broadcasted_iota(jnp.int32, (1, _HW), 1)
    lane_p = jax.lax.broadcasted_iota(jnp.int32, (1, _PPAD), 1)

    def body(t, carry):
        sc, vals, idx = carry
        m = jnp.max(sc, axis=1, keepdims=True)
        i = jnp.min(jnp.where(sc == m, lane, jnp.int32(_HW)),
                    axis=1, keepdims=True)
        vals = jnp.where(lane_p == t, m, vals)
        idx = jnp.where(lane_p == t, i, idx)
        sc = jnp.where(lane == i, -1.0, sc)
        return sc, vals, idx

    carry0 = (sc,
              jnp.zeros((_B, _PPAD), jnp.float32),
              jnp.full((_B, _PPAD), -1, jnp.int32))
    _, vals, idx = jax.lax.fori_loop(0, _P, body, carry0)
    vals_ref[...] = vals
    idx_ref[...] = idx


def _gather_kernel(x_ref, head2_ref, idx_ref, bbox_ref, featsT_ref, loc_ref):
    idxr = idx_ref[0]
    sub = jax.lax.broadcasted_iota(jnp.int32, (_HW, 1), 0)
    onehot_t = (sub == idxr).astype(jnp.float32)

    fsel_t = jax.lax.dot_general(
        x_ref[0], onehot_t, (((1,), (0,)), ((), ())),
        preferred_element_type=jnp.float32)
    dsel = jax.lax.dot_general(
        head2_ref[0], onehot_t, (((1,), (0,)), ((), ())),
        preferred_element_type=jnp.float32)

    xx = idxr % _W
    yy = idxr // _W
    dxv, dyv, dwv, dhv = dsel[1:2], dsel[2:3], dsel[3:4], dsel[4:5]
    cx = xx.astype(jnp.float32) * 8.0 + dxv * 8.0
    cy = yy.astype(jnp.float32) * 8.0 + dyv * 8.0
    wv = jnp.exp(dwv) * 8.0
    hv = jnp.exp(dhv) * 8.0

    bbox_ref[0, 0:1, :] = cx - wv * 0.5
    bbox_ref[0, 1:2, :] = cy - hv * 0.5
    bbox_ref[0, 2:3, :] = cx + wv * 0.5
    bbox_ref[0, 3:4, :] = cy + hv * 0.5
    bbox_ref[0, 4:8, :] = jnp.zeros((4, _PPAD), jnp.float32)
    featsT_ref[0] = fsel_t
    loc_ref[0, 0:1, :] = xx
    loc_ref[0, 1:2, :] = yy
    loc_ref[0, 2:8, :] = jnp.zeros((6, _PPAD), jnp.int32)


def kernel(features, obj_w1, obj_b1, obj_w2, obj_b2,
           box_w1, box_b1, box_w2, box_b2):
    xf = features.reshape(_B, _C, _HW)
    xp = jnp.pad(xf, ((0, 0), (0, 0), (_LPAD, _LPAD)))

    w1c = jnp.concatenate([obj_w1, box_w1], axis=0)
    w1t = jnp.transpose(w1c, (2, 3, 0, 1)).reshape(9, _C, _C)
    b1c = jnp.concatenate([obj_b1, box_b1], axis=0).reshape(_C, 1)
    b1p = jnp.broadcast_to(b1c, (_C, 128))

    w2 = jnp.zeros((8, _C), jnp.float32)
    w2 = w2.at[0, :128].set(obj_w2.reshape(128))
    w2 = w2.at[1:5, 128:].set(box_w2.reshape(4, 128))
    b2 = jnp.zeros((8,), jnp.float32)
    b2 = b2.at[0].set(obj_b2[0]).at[1:5].set(box_b2)
    b2p = jnp.broadcast_to(b2.reshape(8, 1), (8, 128))

    scores3, head2o = pl.pallas_call(
        _heads_kernel,
        grid=(_B,),
        in_specs=[
            pl.BlockSpec((1, _C, _HW + 2 * _LPAD), lambda b: (b, 0, 0)),
            pl.BlockSpec((9, _C, _C), lambda b: (0, 0, 0)),
            pl.BlockSpec((_C, 128), lambda b: (0, 0)),
            pl.BlockSpec((8, _C), lambda b: (0, 0)),
            pl.BlockSpec((8, 128), lambda b: (0, 0)),
        ],
        out_specs=[
            pl.BlockSpec((1, 1, _HW), lambda b: (b, 0, 0)),
            pl.BlockSpec((1, 8, _HW), lambda b: (b, 0, 0)),
        ],
        out_shape=[
            jax.ShapeDtypeStruct((_B, 1, _HW), jnp.float32),
            jax.ShapeDtypeStruct((_B, 8, _HW), jnp.float32),
        ],
    )(xp, w1t, b1p, w2, b2p)

    vals, idx = pl.pallas_call(
        _topk_kernel,
        grid=(1,),
        in_specs=[pl.BlockSpec((_B, 1, _HW), lambda i: (0, 0, 0))],
        out_specs=[
            pl.BlockSpec((_B, _PPAD), lambda i: (0, 0)),
            pl.BlockSpec((_B, _PPAD), lambda i: (0, 0)),
        ],
        out_shape=[
            jax.ShapeDtypeStruct((_B, _PPAD), jnp.float32),
            jax.ShapeDtypeStruct((_B, _PPAD), jnp.int32),
        ],
    )(scores3)

    idx3 = idx.reshape(_B, 1, _PPAD)
    bbox_o, featsT_o, loc_o = pl.pallas_call(
        _gather_kernel,
        grid=(_B,),
        in_specs=[
            pl.BlockSpec((1, _C, _HW), lambda b: (b, 0, 0)),
            pl.BlockSpec((1, 8, _HW), lambda b: (b, 0, 0)),
            pl.BlockSpec((1, 1, _PPAD), lambda b: (b, 0, 0)),
        ],
        out_specs=[
            pl.BlockSpec((1, 8, _PPAD), lambda b: (b, 0, 0)),
            pl.BlockSpec((1, _C, _PPAD), lambda b: (b, 0, 0)),
            pl.BlockSpec((1, 8, _PPAD), lambda b: (b, 0, 0)),
        ],
        out_shape=[
            jax.ShapeDtypeStruct((_B, 8, _PPAD), jnp.float32),
            jax.ShapeDtypeStruct((_B, _C, _PPAD), jnp.float32),
            jax.ShapeDtypeStruct((_B, 8, _PPAD), jnp.int32),
        ],
    )(xf, head2o, idx3)

    top_scores = vals[:, :_P]
    bboxes = bbox_o[:, :4, :_P].transpose(0, 2, 1)
    feats = featsT_o[:, :, :_P].transpose(0, 2, 1)
    locations = loc_o[:, :2, :_P].transpose(0, 2, 1)
    return bboxes, top_scores, feats, locations


# speedup vs baseline: 3.0337x; 1.2602x over previous
import jax
import jax.numpy as jnp
from jax.experimental import pallas as pl
from jax.experimental.pallas import tpu as pltpu

_B, _C, _H, _W = 8, 256, 64, 64
_HW = _H * _W
_P = 100
_PPAD = 128


def _heads_kernel(x_ref, w1_ref, b1_ref, w2_ref, b2_ref,
                  scores_ref, head2_ref):
    x = x_ref[0]
    lane = jax.lax.broadcasted_iota(jnp.int32, (1, _HW), 1)
    xcol = lane % _W
    xm = x * (xcol <= _W - 2).astype(jnp.float32)
    xp_ = x * (xcol >= 1).astype(jnp.float32)
    srcs = {-1: xm, 0: x, 1: xp_}

    acc = jnp.zeros((_C, _HW), jnp.float32)
    for k in range(9):
        dy, dx = k // 3 - 1, k % 3 - 1
        src = srcs[dx]
        off = dy * _W + dx
        if off > 0:
            xs = jnp.concatenate(
                [src[:, off:], jnp.zeros((_C, off), jnp.float32)], axis=1)
        elif off < 0:
            xs = jnp.concatenate(
                [jnp.zeros((_C, -off), jnp.float32), src[:, :off]], axis=1)
        else:
            xs = src
        acc = acc + jax.lax.dot_general(
            w1_ref[k], xs, (((1,), (0,)), ((), ())),
            preferred_element_type=jnp.float32)
    hid = jnp.maximum(acc + b1_ref[:, 0:1], 0.0)

    head2 = jax.lax.dot_general(
        w2_ref[...], hid, (((1,), (0,)), ((), ())),
        preferred_element_type=jnp.float32) + b2_ref[:, 0:1]
    scores_ref[0] = jax.nn.sigmoid(head2[0:1, :])
    head2_ref[0] = head2


def _topk_gather_kernel(scores_ref, x_ref, head2_ref,
                        vals_ref, bbox_ref, featsT_ref, loc_ref,
                        idx_scr):
    b = pl.program_id(0)

    @pl.when(b == 0)
    def _topk():
        sc = scores_ref[:, 0, :]
        lane = jax.lax.broadcasted_iota(jnp.int32, (1, _HW), 1)
        lane_p = jax.lax.broadcasted_iota(jnp.int32, (1, _PPAD), 1)

        def body(t, carry):
            s, vals, idx = carry
            m = jnp.max(s, axis=1, keepdims=True)
            i = jnp.min(jnp.where(s == m, lane, jnp.int32(_HW)),
                        axis=1, keepdims=True)
            vals = jnp.where(lane_p == t, m, vals)
            idx = jnp.where(lane_p == t, i, idx)
            s = jnp.where(lane == i, -1.0, s)
            return s, vals, idx

        carry0 = (sc,
                  jnp.zeros((_B, _PPAD), jnp.float32),
                  jnp.full((_B, _PPAD), -1, jnp.int32))
        _, vals, idx = jax.lax.fori_loop(0, _P, body, carry0)
        vals_ref[...] = vals
        idx_scr[...] = idx

    idxr = idx_scr[pl.ds(b, 1), :]
    sub = jax.lax.broadcasted_iota(jnp.int32, (_HW, 1), 0)
    onehot_t = (sub == idxr).astype(jnp.float32)

    fsel_t = jax.lax.dot_general(
        x_ref[0], onehot_t, (((1,), (0,)), ((), ())),
        preferred_element_type=jnp.float32)
    dsel = jax.lax.dot_general(
        head2_ref[0], onehot_t, (((1,), (0,)), ((), ())),
        preferred_element_type=jnp.float32)

    xx = idxr % _W
    yy = idxr // _W
    dxv, dyv, dwv, dhv = dsel[1:2], dsel[2:3], dsel[3:4], dsel[4:5]
    cx = xx.astype(jnp.float32) * 8.0 + dxv * 8.0
    cy = yy.astype(jnp.float32) * 8.0 + dyv * 8.0
    wv = jnp.exp(dwv) * 8.0
    hv = jnp.exp(dhv) * 8.0

    bbox_ref[0, 0:1, :] = cx - wv * 0.5
    bbox_ref[0, 1:2, :] = cy - hv * 0.5
    bbox_ref[0, 2:3, :] = cx + wv * 0.5
    bbox_ref[0, 3:4, :] = cy + hv * 0.5
    bbox_ref[0, 4:8, :] = jnp.zeros((4, _PPAD), jnp.float32)
    featsT_ref[0] = fsel_t
    loc_ref[0, 0:1, :] = xx
    loc_ref[0, 1:2, :] = yy
    loc_ref[0, 2:8, :] = jnp.zeros((6, _PPAD), jnp.int32)


def kernel(features, obj_w1, obj_b1, obj_w2, obj_b2,
           box_w1, box_b1, box_w2, box_b2):
    xf = features.reshape(_B, _C, _HW)

    w1c = jnp.concatenate([obj_w1, box_w1], axis=0)
    w1t = jnp.transpose(w1c, (2, 3, 0, 1)).reshape(9, _C, _C)
    b1c = jnp.concatenate([obj_b1, box_b1], axis=0).reshape(_C, 1)
    b1p = jnp.broadcast_to(b1c, (_C, 128))

    w2 = jnp.zeros((8, _C), jnp.float32)
    w2 = w2.at[0, :128].set(obj_w2.reshape(128))
    w2 = w2.at[1:5, 128:].set(box_w2.reshape(4, 128))
    b2 = jnp.zeros((8,), jnp.float32)
    b2 = b2.at[0].set(obj_b2[0]).at[1:5].set(box_b2)
    b2p = jnp.broadcast_to(b2.reshape(8, 1), (8, 128))

    scores3, head2o = pl.pallas_call(
        _heads_kernel,
        grid=(_B,),
        in_specs=[
            pl.BlockSpec((1, _C, _HW), lambda b: (b, 0, 0)),
            pl.BlockSpec((9, _C, _C), lambda b: (0, 0, 0)),
            pl.BlockSpec((_C, 128), lambda b: (0, 0)),
            pl.BlockSpec((8, _C), lambda b: (0, 0)),
            pl.BlockSpec((8, 128), lambda b: (0, 0)),
        ],
        out_specs=[
            pl.BlockSpec((1, 1, _HW), lambda b: (b, 0, 0)),
            pl.BlockSpec((1, 8, _HW), lambda b: (b, 0, 0)),
        ],
        out_shape=[
            jax.ShapeDtypeStruct((_B, 1, _HW), jnp.float32),
            jax.ShapeDtypeStruct((_B, 8, _HW), jnp.float32),
        ],
    )(xf, w1t, b1p, w2, b2p)

    vals, bbox_o, featsT_o, loc_o = pl.pallas_call(
        _topk_gather_kernel,
        grid=(_B,),
        in_specs=[
            pl.BlockSpec((_B, 1, _HW), lambda b: (0, 0, 0)),
            pl.BlockSpec((1, _C, _HW), lambda b: (b, 0, 0)),
            pl.BlockSpec((1, 8, _HW), lambda b: (b, 0, 0)),
        ],
        out_specs=[
            pl.BlockSpec((_B, _PPAD), lambda b: (0, 0)),
            pl.BlockSpec((1, 8, _PPAD), lambda b: (b, 0, 0)),
            pl.BlockSpec((1, _C, _PPAD), lambda b: (b, 0, 0)),
            pl.BlockSpec((1, 8, _PPAD), lambda b: (b, 0, 0)),
        ],
        out_shape=[
            jax.ShapeDtypeStruct((_B, _PPAD), jnp.float32),
            jax.ShapeDtypeStruct((_B, 8, _PPAD), jnp.float32),
            jax.ShapeDtypeStruct((_B, _C, _PPAD), jnp.float32),
            jax.ShapeDtypeStruct((_B, 8, _PPAD), jnp.int32),
        ],
        scratch_shapes=[pltpu.VMEM((_B, _PPAD), jnp.int32)],
    )(scores3, xf, head2o)

    top_scores = vals[:, :_P]
    bboxes = bbox_o[:, :4, :_P].transpose(0, 2, 1)
    feats = featsT_o[:, :, :_P].transpose(0, 2, 1)
    locations = loc_o[:, :2, :_P].transpose(0, 2, 1)
    return bboxes, top_scores, feats, locations
